# baseline (speedup 1.0000x reference)
import functools

import jax
import jax.numpy as jnp
from jax import lax
from jax.experimental import pallas as pl
from jax.experimental.pallas import tpu as pltpu
from jax.experimental.pallas import tpu_sc as plsc

N_ROWS = 64
N_COLS = 1_000_000
NC = 2
NS = 16
NW = NC * NS
ROWS_PER_W = N_ROWS // NW

L = 16
CHUNK = 50_000
NCHUNK = N_COLS // CHUNK
UNROLL = 25
VECS = CHUNK // L
N_IN = VECS // UNROLL
BIG = 2**30
NEG_INF = float("-inf")


def _lane_reduce(vec, op):
    vals = [vec[i] for i in range(L)]
    while len(vals) > 1:
        vals = [op(vals[i], vals[i + 1]) for i in range(0, len(vals), 2)]
    return vals[0]


def _chunk_max(buf, gbase):
    del gbase

    def body(j, cv):
        base = j * (UNROLL * L)
        for u in range(UNROLL):
            v = buf[pl.ds(base + u * L, L)]
            cv = jnp.maximum(cv, v)
        return cv

    cv0 = jnp.full((L,), NEG_INF, dtype=jnp.float32)
    return lax.fori_loop(0, N_IN, body, cv0, unroll=False)


def _first_pos_of(buf, gmax):
    iota = lax.iota(jnp.int32, L)
    gvec = jnp.full((L,), gmax, dtype=jnp.float32)

    def body(j, rm):
        base = j * (UNROLL * L)
        for u in range(UNROLL):
            off = base + u * L
            v = buf[pl.ds(off, L)]
            pos = iota + off
            rm = jnp.minimum(rm, jnp.where(v == gvec, pos, BIG))
        return rm

    rm0 = jnp.full((L,), BIG, dtype=jnp.int32)
    rm = lax.fori_loop(0, N_IN, body, rm0, unroll=False)
    return _lane_reduce(rm, jnp.minimum)


def _body(in_hbm, out_hbm, buf0, buf1, cmax, stage, sem0, sem1):
    wid = lax.axis_index("c") * NS + lax.axis_index("s")
    iota = lax.iota(jnp.int32, L)
    row_idx = []

    for r in range(ROWS_PER_W):
        row_base = (wid * ROWS_PER_W + r) * N_COLS

        def start(c_idx, tgt, sem):
            off = pl.multiple_of(row_base + c_idx * CHUNK, CHUNK)
            return pltpu.async_copy(in_hbm.at[pl.ds(off, CHUNK)], tgt, sem)

        start(0, buf0, sem0)
        start(1, buf1, sem1)

        def chunk_pair(i, _):
            pltpu.make_async_copy(in_hbm.at[pl.ds(0, CHUNK)], buf0, sem0).wait()
            cv0 = _chunk_max(buf0, None)
            cmax[pl.ds((2 * i) * L, L)] = cv0

            @pl.when(i < NCHUNK // 2 - 1)
            def _():
                start(2 * i + 2, buf0, sem0)

            pltpu.make_async_copy(in_hbm.at[pl.ds(0, CHUNK)], buf1, sem1).wait()
            cv1 = _chunk_max(buf1, None)
            cmax[pl.ds((2 * i + 1) * L, L)] = cv1

            @pl.when(i < NCHUNK // 2 - 1)
            def _():
                start(2 * i + 3, buf1, sem1)

            return 0

        lax.fori_loop(0, NCHUNK // 2, chunk_pair, 0, unroll=False)

        def gbody(k, gv):
            return jnp.maximum(gv, cmax[pl.ds(k * L, L)])

        gvec = lax.fori_loop(0, NCHUNK, gbody,
                             jnp.full((L,), NEG_INF, dtype=jnp.float32),
                             unroll=False)
        gmax = _lane_reduce(gvec, jnp.maximum)
        gsplat = jnp.full((L,), gmax, dtype=jnp.float32)

        def kbody(k, kv):
            m = cmax[pl.ds(k * L, L)] == gsplat
            return jnp.minimum(kv, jnp.where(m, jnp.full((L,), 0, jnp.int32) + k, BIG))

        kvec = lax.fori_loop(0, NCHUNK, kbody,
                             jnp.full((L,), BIG, dtype=jnp.int32),
                             unroll=False)
        kwin = _lane_reduce(kvec, jnp.minimum)

        off = pl.multiple_of(row_base + kwin * CHUNK, L)
        pltpu.async_copy(in_hbm.at[pl.ds(off, CHUNK)], buf0, sem0).wait()
        row_idx.append(kwin * CHUNK + _first_pos_of(buf0, gmax))

    res = jnp.full((L,), 0, dtype=jnp.int32)
    for r in range(ROWS_PER_W):
        res = jnp.where(iota == r, jnp.full((L,), row_idx[r], jnp.int32), res)
    stage[...] = res
    pltpu.sync_copy(stage, out_hbm.at[wid])


@functools.partial(jax.jit, static_argnames=())
def _argmax_rows(flat):
    mesh = plsc.VectorSubcoreMesh(core_axis_name="c", subcore_axis_name="s")
    kern = pl.kernel(
        _body,
        out_type=jax.ShapeDtypeStruct((NW, L), jnp.int32),
        mesh=mesh,
        scratch_types=[
            pltpu.VMEM((CHUNK,), jnp.float32),
            pltpu.VMEM((CHUNK,), jnp.float32),
            pltpu.VMEM((NCHUNK * L,), jnp.float32),
            pltpu.VMEM((L,), jnp.int32),
            pltpu.SemaphoreType.DMA,
            pltpu.SemaphoreType.DMA,
        ],
    )
    return kern(flat)


def kernel(inputs):
    flat = inputs.reshape(-1)
    out2d = _argmax_rows(flat)
    return out2d[:, :ROWS_PER_W].reshape(N_ROWS).astype(jnp.int64)


# speedup vs baseline: 1.0010x; 1.0010x over previous
import functools

import jax
import jax.numpy as jnp
from jax import lax
from jax.experimental import pallas as pl
from jax.experimental.pallas import tpu as pltpu
from jax.experimental.pallas import tpu_sc as plsc

N_ROWS = 64
N_COLS = 1_000_000
NC = 2
NS = 16
NW = NC * NS
ROWS_PER_W = N_ROWS // NW

L = 16
CHUNK = 50_000
NCHUNK = N_COLS // CHUNK
UNROLL = 25
VECS = CHUNK // L
N_IN = VECS // UNROLL
BIG = 2**30
NEG_INF = float("-inf")


def _lane_reduce(vec, op):
    vals = [vec[i] for i in range(L)]
    while len(vals) > 1:
        vals = [op(vals[i], vals[i + 1]) for i in range(0, len(vals), 2)]
    return vals[0]


def _tree_combine(vals, op):
    vals = list(vals)
    while len(vals) > 1:
        nxt = [op(vals[i], vals[i + 1]) for i in range(0, len(vals) - 1, 2)]
        if len(vals) % 2:
            nxt.append(vals[-1])
        vals = nxt
    return vals[0]


def _chunk_max(buf):
    accs0 = tuple(jnp.full((L,), NEG_INF, dtype=jnp.float32)
                  for _ in range(UNROLL))

    @plsc.parallel_loop(0, VECS, step=UNROLL, carry=accs0)
    def body(i, accs):
        base = i * L
        return tuple(
            jnp.maximum(accs[u], buf[pl.ds(base + u * L, L)])
            for u in range(UNROLL)
        )

    return _tree_combine(body, jnp.maximum)


def _first_pos_of(buf, gmax):
    iota = lax.iota(jnp.int32, L)
    gvec = jnp.full((L,), gmax, dtype=jnp.float32)

    rms0 = tuple(jnp.full((L,), BIG, dtype=jnp.int32) for _ in range(UNROLL))

    @plsc.parallel_loop(0, VECS, step=UNROLL, carry=rms0)
    def body(i, rms):
        base = i * L
        out = []
        for u in range(UNROLL):
            off = base + u * L
            v = buf[pl.ds(off, L)]
            out.append(jnp.minimum(rms[u], jnp.where(v == gvec, iota + off, BIG)))
        return tuple(out)

    rm = _tree_combine(body, jnp.minimum)
    return _lane_reduce(rm, jnp.minimum)


def _body(in_hbm, out_hbm, buf0, buf1, cmax, stage, sem0, sem1):
    wid = lax.axis_index("c") * NS + lax.axis_index("s")
    iota = lax.iota(jnp.int32, L)
    row_idx = []

    for r in range(ROWS_PER_W):
        row_base = (wid * ROWS_PER_W + r) * N_COLS

        def start(c_idx, tgt, sem):
            off = pl.multiple_of(row_base + c_idx * CHUNK, CHUNK)
            return pltpu.async_copy(in_hbm.at[pl.ds(off, CHUNK)], tgt, sem)

        start(0, buf0, sem0)
        start(1, buf1, sem1)

        def chunk_pair(i, _):
            pltpu.make_async_copy(in_hbm.at[pl.ds(0, CHUNK)], buf0, sem0).wait()
            cv0 = _chunk_max(buf0)
            cmax[pl.ds((2 * i) * L, L)] = cv0

            @pl.when(i < NCHUNK // 2 - 1)
            def _():
                start(2 * i + 2, buf0, sem0)

            pltpu.make_async_copy(in_hbm.at[pl.ds(0, CHUNK)], buf1, sem1).wait()
            cv1 = _chunk_max(buf1)
            cmax[pl.ds((2 * i + 1) * L, L)] = cv1

            @pl.when(i < NCHUNK // 2 - 1)
            def _():
                start(2 * i + 3, buf1, sem1)

            return 0

        lax.fori_loop(0, NCHUNK // 2, chunk_pair, 0, unroll=False)

        def gbody(k, gv):
            return jnp.maximum(gv, cmax[pl.ds(k * L, L)])

        gvec = lax.fori_loop(0, NCHUNK, gbody,
                             jnp.full((L,), NEG_INF, dtype=jnp.float32),
                             unroll=False)
        gmax = _lane_reduce(gvec, jnp.maximum)
        gsplat = jnp.full((L,), gmax, dtype=jnp.float32)

        def kbody(k, kv):
            m = cmax[pl.ds(k * L, L)] == gsplat
            return jnp.minimum(kv, jnp.where(m, jnp.full((L,), 0, jnp.int32) + k, BIG))

        kvec = lax.fori_loop(0, NCHUNK, kbody,
                             jnp.full((L,), BIG, dtype=jnp.int32),
                             unroll=False)
        kwin = _lane_reduce(kvec, jnp.minimum)

        off = pl.multiple_of(row_base + kwin * CHUNK, L)
        pltpu.async_copy(in_hbm.at[pl.ds(off, CHUNK)], buf0, sem0).wait()
        row_idx.append(kwin * CHUNK + _first_pos_of(buf0, gmax))

    res = jnp.full((L,), 0, dtype=jnp.int32)
    for r in range(ROWS_PER_W):
        res = jnp.where(iota == r, jnp.full((L,), row_idx[r], jnp.int32), res)
    stage[...] = res
    pltpu.sync_copy(stage, out_hbm.at[wid])


@functools.partial(jax.jit, static_argnames=())
def _argmax_rows(flat):
    mesh = plsc.VectorSubcoreMesh(core_axis_name="c", subcore_axis_name="s")
    kern = pl.kernel(
        _body,
        out_type=jax.ShapeDtypeStruct((NW, L), jnp.int32),
        mesh=mesh,
        scratch_types=[
            pltpu.VMEM((CHUNK,), jnp.float32),
            pltpu.VMEM((CHUNK,), jnp.float32),
            pltpu.VMEM((NCHUNK * L,), jnp.float32),
            pltpu.VMEM((L,), jnp.int32),
            pltpu.SemaphoreType.DMA,
            pltpu.SemaphoreType.DMA,
        ],
    )
    return kern(flat)


def kernel(inputs):
    flat = inputs.reshape(-1)
    out2d = _argmax_rows(flat)
    return out2d[:, :ROWS_PER_W].reshape(N_ROWS).astype(jnp.int64)


# speedup vs baseline: 34.2166x; 34.1813x over previous
import jax
import jax.numpy as jnp
from jax import lax
from jax.experimental import pallas as pl
from jax.experimental.pallas import tpu as pltpu
from jax.experimental.pallas import tpu_sc as plsc

N_ROWS = 64
N_COLS = 1_000_000
NC = 2
NS = 16
L = 16

TILE_R = 8
TILE_C = 128
NTR = N_ROWS // TILE_R
NQ = 4

FULL_TILES = N_COLS // TILE_C
TPQ = FULL_TILES // NQ
CQ = TPQ * TILE_C
TAIL_C = N_COLS - FULL_TILES * TILE_C
TAIL_BASE = FULL_TILES * TILE_C

CT = 31
CW = CT * TILE_C
NCH = TPQ // CT
NREC = NCH + 1
VPT = TILE_C // L

BIG = 2**30
NEG_INF = float("-inf")


def _lane_reduce(vec, op):
    vals = [vec[i] for i in range(L)]
    while len(vals) > 1:
        vals = [op(vals[i], vals[i + 1]) for i in range(0, len(vals), 2)]
    return vals[0]


def _window_max(buf):
    accs0 = tuple(jnp.full((L,), NEG_INF, dtype=jnp.float32)
                  for _ in range(TILE_R))

    @plsc.parallel_loop(0, CT, step=1, carry=accs0)
    def body(t, accs):
        ct = pl.multiple_of(t * TILE_C, TILE_C)
        out = list(accs)
        for r in range(TILE_R):
            for h in range(VPT):
                out[r] = jnp.maximum(out[r], buf[r, pl.ds(ct + h * L, L)])
        return tuple(out)

    return body


def _row_first_pos(buf, r, gmax, col0):
    iota = lax.iota(jnp.int32, L)
    gvec = jnp.full((L,), gmax, dtype=jnp.float32)

    rms0 = tuple(jnp.full((L,), BIG, dtype=jnp.int32) for _ in range(VPT))

    @plsc.parallel_loop(0, CT, step=1, carry=rms0)
    def body(t, rms):
        ct = pl.multiple_of(t * TILE_C, TILE_C)
        base = col0 + t * TILE_C
        out = []
        for h in range(VPT):
            v = buf[r, pl.ds(ct + h * L, L)]
            pos = iota + (base + h * L)
            out.append(jnp.minimum(rms[h], jnp.where(v == gvec, pos, BIG)))
        return tuple(out)

    rm = body[0]
    for h in range(1, VPT):
        rm = jnp.minimum(rm, body[h])
    return _lane_reduce(rm, jnp.minimum)


def _tail_scan(btail, r, gmax):
    iota = lax.iota(jnp.int32, L)
    gvec = jnp.full((L,), gmax, dtype=jnp.float32)
    rm = jnp.full((L,), BIG, dtype=jnp.int32)
    for h in range(TAIL_C // L):
        v = btail[pl.ds(r * TAIL_C + h * L, L)]
        pos = iota + (TAIL_BASE + h * L)
        rm = jnp.minimum(rm, jnp.where(v == gvec, pos, BIG))
    return _lane_reduce(rm, jnp.minimum)


def _body(in_hbm, tail_hbm, out_hbm,
          buf0, buf1, btail, cmax, vstage, istage, tmpf, tmpi,
          shv, shi, sem0, sem1):
    c = lax.axis_index("c")
    s = lax.axis_index("s")
    tr = c * (NTR // NC) + s // NQ
    q = s % NQ
    iota = lax.iota(jnp.int32, L)

    row0 = pl.multiple_of(tr * TILE_R, TILE_R)
    cb = pl.multiple_of(q * CQ, TILE_C)

    def start(k, tgt, sem):
        off = pl.multiple_of(cb + k * CW, TILE_C)
        return pltpu.async_copy(
            in_hbm.at[pl.ds(row0, TILE_R), pl.ds(off, CW)], tgt, sem)

    def wait(tgt, sem):
        pltpu.make_async_copy(
            in_hbm.at[pl.ds(0, TILE_R), pl.ds(0, CW)], tgt, sem).wait()

    def record(k, accs):
        for r in range(TILE_R):
            cmax[pl.ds((k * TILE_R + r) * L, L)] = accs[r]

    pltpu.sync_copy(tail_hbm.at[pl.ds(row0 * TAIL_C, TILE_R * TAIL_C)], btail)

    ninf = jnp.full((L,), NEG_INF, dtype=jnp.float32)
    record(NCH, (ninf,) * TILE_R)

    start(0, buf0, sem0)
    start(1, buf1, sem1)

    def chunk_pair(i, _):
        wait(buf0, sem0)
        a0 = _window_max(buf0)
        record(2 * i, a0)

        @pl.when(2 * i + 2 < NCH)
        def _():
            start(2 * i + 2, buf0, sem0)

        wait(buf1, sem1)
        a1 = _window_max(buf1)
        record(2 * i + 1, a1)

        @pl.when(2 * i + 3 < NCH)
        def _():
            start(2 * i + 3, buf1, sem1)

        return 0

    lax.fori_loop(0, NCH // 2, chunk_pair, 0, unroll=False)
    wait(buf0, sem0)
    record(NCH - 1, _window_max(buf0))

    @pl.when(q == NQ - 1)
    def _():
        for r in range(TILE_R):
            mr = jnp.full((L,), NEG_INF, dtype=jnp.float32)
            for h in range(TAIL_C // L):
                mr = jnp.maximum(mr, btail[pl.ds(r * TAIL_C + h * L, L)])
            cmax[pl.ds((NCH * TILE_R + r) * L, L)] = mr

    lvals = []
    lidxs = []
    for r in range(TILE_R):
        def gbody(k, gv, r=r):
            return jnp.maximum(gv, cmax[pl.ds((k * TILE_R + r) * L, L)])

        gvec = lax.fori_loop(0, NREC, gbody,
                             jnp.full((L,), NEG_INF, dtype=jnp.float32),
                             unroll=False)
        gmax = _lane_reduce(gvec, jnp.maximum)
        gsplat = jnp.full((L,), gmax, dtype=jnp.float32)

        def kbody(k, kv, r=r, gsplat=gsplat):
            m = cmax[pl.ds((k * TILE_R + r) * L, L)] == gsplat
            return jnp.minimum(kv, jnp.where(m, jnp.zeros((L,), jnp.int32) + k, BIG))

        kvec = lax.fori_loop(0, NREC, kbody,
                             jnp.full((L,), BIG, dtype=jnp.int32),
                             unroll=False)
        kwin = _lane_reduce(kvec, jnp.minimum)

        kcl = jnp.minimum(kwin, NCH - 1)
        start(kcl, buf0, sem0).wait()
        pos_main = _row_first_pos(buf0, r, gmax, cb + kcl * CW)
        pos_tail = _tail_scan(btail, r, gmax)
        lvals.append(gmax)
        lidxs.append(jnp.where(kwin == NCH, pos_tail, pos_main))

    lval = jnp.full((L,), NEG_INF, dtype=jnp.float32)
    lidx = jnp.zeros((L,), jnp.int32) + BIG
    for r in range(TILE_R):
        lval = jnp.where(iota == r, jnp.full((L,), lvals[r], jnp.float32), lval)
        lidx = jnp.where(iota == r, jnp.full((L,), lidxs[r], jnp.int32), lidx)

    vstage[...] = lval
    istage[...] = lidx
    pltpu.sync_copy(vstage, shv.at[pl.ds(s * L, L)])
    pltpu.sync_copy(istage, shi.at[pl.ds(s * L, L)])
    plsc.subcore_barrier()

    @pl.when(q == 0)
    def _():
        bestv = lval
        besti = lidx
        for peer in range(1, NQ):
            pltpu.sync_copy(shv.at[pl.ds((s + peer) * L, L)], tmpf)
            pltpu.sync_copy(shi.at[pl.ds((s + peer) * L, L)], tmpi)
            pv = tmpf[...]
            pi = tmpi[...]
            take = (pv > bestv) | ((pv == bestv) & (pi < besti))
            bestv = jnp.where(take, pv, bestv)
            besti = jnp.where(take, pi, besti)
        istage[...] = besti
        pltpu.sync_copy(istage, out_hbm.at[tr])


@jax.jit
def _argmax_rows(x2d, tail):
    mesh = plsc.VectorSubcoreMesh(core_axis_name="c", subcore_axis_name="s")
    kern = pl.kernel(
        _body,
        out_type=jax.ShapeDtypeStruct((NTR, L), jnp.int32),
        mesh=mesh,
        compiler_params=pltpu.CompilerParams(use_tc_tiling_on_sc=True),
        scratch_types=[
            pltpu.VMEM((TILE_R, CW), jnp.float32),
            pltpu.VMEM((TILE_R, CW), jnp.float32),
            pltpu.VMEM((TILE_R * TAIL_C,), jnp.float32),
            pltpu.VMEM((NREC * TILE_R * L,), jnp.float32),
            pltpu.VMEM((L,), jnp.float32),
            pltpu.VMEM((L,), jnp.int32),
            pltpu.VMEM((L,), jnp.float32),
            pltpu.VMEM((L,), jnp.int32),
            pltpu.VMEM_SHARED((NS * L,), jnp.float32),
            pltpu.VMEM_SHARED((NS * L,), jnp.int32),
            pltpu.SemaphoreType.DMA,
            pltpu.SemaphoreType.DMA,
        ],
    )
    return kern(x2d, tail)


def kernel(inputs):
    tail = inputs[:, TAIL_BASE:].reshape(-1)
    out2d = _argmax_rows(inputs, tail)
    return out2d[:, :TILE_R].reshape(N_ROWS).astype(jnp.int64)
